# baseline (speedup 1.0000x reference)
import functools

import jax
import jax.numpy as jnp
import numpy as np
from jax import lax
from jax.experimental import pallas as pl
from jax.experimental.pallas import tpu as pltpu
from jax.experimental.pallas import tpu_sc as plsc

_B, _N, _E, _H, _K = 8, 2048, 256, 8, 8
_HD = _E // _H

_F32 = jnp.float32
_HIGH = lax.Precision.HIGHEST
_DEF = lax.Precision.DEFAULT

_TNB = 512


def _knn_body(xvr_ref, xvc_ref, out_ref):
    b = pl.program_id(0)
    xr = xvr_ref[0]
    xc = xvc_ref[0]
    sqr = jnp.sum(xr * xr, axis=1, keepdims=True)
    ones8 = jnp.ones((8, 128), _F32)
    sqc = lax.dot_general(ones8, xc * xc, (((1,), (1,)), ((), ())),
                          preferred_element_type=_F32, precision=_HIGH)
    sqc_row = sqc[0:1, :]
    g = lax.dot_general(xr, xc, (((1,), (1,)), ((), ())),
                        preferred_element_type=_F32,
                        precision=lax.Precision.DEFAULT)
    d = (sqr + sqc_row) - 2.0 * g
    iota = lax.broadcasted_iota(jnp.int32, (_TNB, _N), 1)
    u = lax.bitcast_convert_type(jnp.maximum(d, 0.0), jnp.int32)
    key_i = (u & jnp.int32(~2047)) | iota
    key = lax.bitcast_convert_type(key_i, _F32)
    for t in range(_K):
        m = jnp.min(key, axis=1, keepdims=True)
        mi = lax.bitcast_convert_type(m[:, 0], jnp.int32)
        out_ref[0, t, :] = (mi & 2047) + b * _N
        key = jnp.where(key == m, jnp.float32(jnp.inf), key)


def _knn_idx_global(xvp):
    return pl.pallas_call(
        _knn_body,
        grid=(_B, _N // _TNB),
        in_specs=[
            pl.BlockSpec((1, _TNB, 128), lambda b, i: (b, i, 0)),
            pl.BlockSpec((1, _N, 128), lambda b, i: (b, 0, 0)),
        ],
        out_specs=pl.BlockSpec((1, _K, _TNB), lambda b, i: (b, 0, i)),
        out_shape=jax.ShapeDtypeStruct((_B, _K, _N), jnp.int32),
    )(xvp, xvp)


_TNP = 512


def _proj_body(x_ref, w_ref, out_ref):
    pf = jnp.dot(x_ref[:], w_ref[:],
                 preferred_element_type=_F32, precision=_DEF)
    ki = lax.bitcast_convert_type(pf[:, :_E], jnp.int32)
    vi = lax.bitcast_convert_type(pf[:, _E:], jnp.int32)
    kr = ki + 0x7FFF + ((ki >> 16) & 1)
    vr = vi + 0x7FFF + ((vi >> 16) & 1)
    out_ref[:] = (lax.shift_right_logical(kr, 16) & 0xFFFF) | \
        (vr & jnp.int32(-65536))


def _proj_kv(x2, wkvT):
    return pl.pallas_call(
        _proj_body,
        grid=(_B * _N // _TNP,),
        in_specs=[
            pl.BlockSpec((_TNP, _E), lambda i: (i, 0)),
            pl.BlockSpec((_E, 2 * _E), lambda i: (0, 0)),
        ],
        out_specs=pl.BlockSpec((_TNP, _E), lambda i: (i, 0)),
        out_shape=jax.ShapeDtypeStruct((_B * _N, _E), jnp.int32),
    )(x2, wkvT)


_SC_CHUNK = 128
_ROWS = _B * _N * _K


def _gather_rows_sc(xkv, gidx):
    info = plsc.get_sparse_core_info()
    nw = info.num_cores * info.num_subcores
    rpw = _ROWS // nw
    nch = rpw // _SC_CHUNK
    mesh = plsc.VectorSubcoreMesh(core_axis_name="c", subcore_axis_name="s")

    @functools.partial(
        pl.kernel,
        mesh=mesh,
        out_type=jax.ShapeDtypeStruct((_ROWS, _E), jnp.int32),
        scratch_types=[
            pltpu.VMEM((rpw,), jnp.int32),
            pltpu.VMEM((_SC_CHUNK, _E), jnp.int32),
            pltpu.VMEM((_SC_CHUNK, _E), jnp.int32),
            pltpu.SemaphoreType.DMA,
            pltpu.SemaphoreType.DMA,
            pltpu.SemaphoreType.DMA,
            pltpu.SemaphoreType.DMA,
        ],
    )
    def k(xkv_hbm, gidx_hbm, out_hbm, idx_all, buf0, buf1,
          g0, g1, w0, w1):
        wid = lax.axis_index("s") * info.num_cores + lax.axis_index("c")
        base0 = wid * rpw
        pltpu.sync_copy(gidx_hbm.at[pl.ds(base0, rpw)], idx_all)
        bufs, gsem, wsem = (buf0, buf1), (g0, g1), (w0, w1)
        ghs = [None, None]
        whs = [None, None]
        for i in range(nch):
            p = i % 2
            if whs[p] is not None:
                whs[p].wait()
            ghs[p] = pltpu.async_copy(
                xkv_hbm.at[idx_all.at[pl.ds(i * _SC_CHUNK, _SC_CHUNK)]],
                bufs[p], gsem[p])
            q = 1 - p
            if ghs[q] is not None:
                ghs[q].wait()
                whs[q] = pltpu.async_copy(
                    bufs[q],
                    out_hbm.at[pl.ds(base0 + (i - 1) * _SC_CHUNK, _SC_CHUNK)],
                    wsem[q])
        pl2 = (nch - 1) % 2
        ghs[pl2].wait()
        pltpu.async_copy(
            bufs[pl2],
            out_hbm.at[pl.ds(base0 + (nch - 1) * _SC_CHUNK, _SC_CHUNK)],
            wsem[pl2]).wait()
        whs[1 - pl2].wait()

    return k(xkv, gidx)


_TNA = 256


def _attn_body(x_ref, xkv_ref, kvp_ref, wqT_ref, woT_ref, out_ref):
    xr = x_ref[:]
    q = jnp.dot(xr, wqT_ref[:], preferred_element_type=_F32, precision=_DEF)
    pk = kvp_ref[:]
    kp3 = lax.bitcast_convert_type(pk << 16, _F32)
    vp3 = lax.bitcast_convert_type(pk & jnp.int32(-65536), _F32)
    lane_h = lax.broadcasted_iota(jnp.int32, (_E, _H), 0) // _HD
    head_h = lax.broadcasted_iota(jnp.int32, (_E, _H), 1)
    hm = (lane_h == head_h).astype(_F32)
    lane_v = lax.broadcasted_iota(jnp.int32, (_H, _E), 1) // _HD
    head_v = lax.broadcasted_iota(jnp.int32, (_H, _E), 0)
    hmT = (lane_v == head_v).astype(_F32)

    s = kp3 * q[:, None, :]
    scores = jnp.dot(s.reshape(_TNA * _K, _E), hm,
                     preferred_element_type=_F32, precision=_DEF)
    scores = scores / np.sqrt(_HD).astype(np.float32)
    sc3 = scores.reshape(_TNA, _K, _H)
    mx = jnp.max(sc3, axis=1, keepdims=True)
    e = jnp.exp(sc3 - mx)
    a = e / jnp.sum(e, axis=1, keepdims=True)
    attn_exp = jnp.dot(a.reshape(_TNA * _K, _H), hmT,
                       preferred_element_type=_F32, precision=_DEF)
    o = jnp.sum(attn_exp.reshape(_TNA, _K, _E) * vp3, axis=1)
    o = o - lax.bitcast_convert_type(
        xkv_ref[:] & jnp.int32(-65536), _F32)
    out_ref[:] = xr + jnp.dot(o, woT_ref[:],
                              preferred_element_type=_F32, precision=_DEF)


def _attn_out(x2, xkv, kvp3, wqT, woutT):
    nrows = x2.shape[0]
    return pl.pallas_call(
        _attn_body,
        grid=(nrows // _TNA,),
        in_specs=[
            pl.BlockSpec((_TNA, _E), lambda i: (i, 0)),
            pl.BlockSpec((_TNA, _E), lambda i: (i, 0)),
            pl.BlockSpec((_TNA, _K, _E), lambda i: (i, 0, 0)),
            pl.BlockSpec((_E, _E), lambda i: (0, 0)),
            pl.BlockSpec((_E, _E), lambda i: (0, 0)),
        ],
        out_specs=pl.BlockSpec((_TNA, _E), lambda i: (i, 0)),
        out_shape=jax.ShapeDtypeStruct((nrows, _E), _F32),
    )(x2, xkv, kvp3, wqT, woutT)


def kernel(x, x_v, in_proj_weight, out_proj_weight):
    x2 = x.reshape(_B * _N, _E)
    xvp = jnp.pad(x_v, ((0, 0), (0, 0), (0, 128 - 3)))
    Wq, Wk, Wv = jnp.split(in_proj_weight, 3, axis=0)
    wqT = Wq.T
    wkvT = jnp.concatenate([Wk.T, Wv.T], axis=1)
    woutT = out_proj_weight.T

    gidxT = _knn_idx_global(xvp)
    gidx = jnp.transpose(gidxT, (0, 2, 1)).reshape(_ROWS)

    xkv = _proj_kv(x2, wkvT)
    kvp = _gather_rows_sc(xkv, gidx)
    out2 = _attn_out(x2, xkv, kvp.reshape(_B * _N, _K, _E), wqT, woutT)
    return out2.reshape(_B, _N, _E)


# speedup vs baseline: 1.1367x; 1.1367x over previous
import functools

import jax
import jax.numpy as jnp
import numpy as np
from jax import lax
from jax.experimental import pallas as pl
from jax.experimental.pallas import tpu as pltpu
from jax.experimental.pallas import tpu_sc as plsc

_B, _N, _E, _H, _K = 8, 2048, 256, 8, 8
_HD = _E // _H

_F32 = jnp.float32
_HIGH = lax.Precision.HIGHEST
_DEF = lax.Precision.DEFAULT

_TNB = 512


def _knn_body(xvr_ref, xvc_ref, out_ref):
    b = pl.program_id(0)
    xr = xvr_ref[0]
    xc = xvc_ref[0]
    sqr = jnp.sum(xr * xr, axis=1, keepdims=True)
    ones8 = jnp.ones((8, 128), _F32)
    sqc = lax.dot_general(ones8, xc * xc, (((1,), (1,)), ((), ())),
                          preferred_element_type=_F32, precision=_HIGH)
    sqc_row = sqc[0:1, :]
    g = lax.dot_general(xr, xc, (((1,), (1,)), ((), ())),
                        preferred_element_type=_F32,
                        precision=lax.Precision.DEFAULT)
    d = (sqr + sqc_row) - 2.0 * g
    iota = lax.broadcasted_iota(jnp.int32, (_TNB, _N), 1)
    u = lax.bitcast_convert_type(jnp.maximum(d, 0.0), jnp.int32)
    key_i = (u & jnp.int32(~2047)) | iota
    key = lax.bitcast_convert_type(key_i + jnp.int32(0x00800000), _F32)
    for t in range(_K):
        m = jnp.min(key, axis=1, keepdims=True)
        mi = lax.bitcast_convert_type(m[:, 0], jnp.int32)
        out_ref[0, t, :] = (mi & 2047) + b * _N
        key = jnp.where(key == m, jnp.float32(jnp.inf), key)


def _knn_idx_global(xvp):
    return pl.pallas_call(
        _knn_body,
        grid=(_B, _N // _TNB),
        in_specs=[
            pl.BlockSpec((1, _TNB, 128), lambda b, i: (b, i, 0)),
            pl.BlockSpec((1, _N, 128), lambda b, i: (b, 0, 0)),
        ],
        out_specs=pl.BlockSpec((1, _K, _TNB), lambda b, i: (b, 0, i)),
        out_shape=jax.ShapeDtypeStruct((_B, _K, _N), jnp.int32),
    )(xvp, xvp)


_TNP = 512


def _proj_body(x_ref, w_ref, out_ref):
    pf = jnp.dot(x_ref[:], w_ref[:],
                 preferred_element_type=_F32, precision=_DEF)
    ki = lax.bitcast_convert_type(pf[:, :_E], jnp.int32)
    vi = lax.bitcast_convert_type(pf[:, _E:], jnp.int32)
    kr = ki + 0x7FFF + ((ki >> 16) & 1)
    vr = vi + 0x7FFF + ((vi >> 16) & 1)
    out_ref[:] = (lax.shift_right_logical(kr, 16) & 0xFFFF) | \
        (vr & jnp.int32(-65536))


def _proj_kv(x2, wkvT):
    return pl.pallas_call(
        _proj_body,
        grid=(_B * _N // _TNP,),
        in_specs=[
            pl.BlockSpec((_TNP, _E), lambda i: (i, 0)),
            pl.BlockSpec((_E, 2 * _E), lambda i: (0, 0)),
        ],
        out_specs=pl.BlockSpec((_TNP, _E), lambda i: (i, 0)),
        out_shape=jax.ShapeDtypeStruct((_B * _N, _E), jnp.int32),
    )(x2, wkvT)


_SC_CHUNK = 128
_ROWS = _B * _N * _K


def _gather_rows_sc(xkv, gidx):
    info = plsc.get_sparse_core_info()
    nw = info.num_cores * info.num_subcores
    rpw = _ROWS // nw
    nch = rpw // _SC_CHUNK
    mesh = plsc.VectorSubcoreMesh(core_axis_name="c", subcore_axis_name="s")

    @functools.partial(
        pl.kernel,
        mesh=mesh,
        out_type=jax.ShapeDtypeStruct((_ROWS, _E), jnp.int32),
        scratch_types=[
            pltpu.VMEM((rpw,), jnp.int32),
            pltpu.VMEM((_SC_CHUNK, _E), jnp.int32),
            pltpu.VMEM((_SC_CHUNK, _E), jnp.int32),
            pltpu.SemaphoreType.DMA,
            pltpu.SemaphoreType.DMA,
            pltpu.SemaphoreType.DMA,
            pltpu.SemaphoreType.DMA,
        ],
    )
    def k(xkv_hbm, gidx_hbm, out_hbm, idx_all, buf0, buf1,
          g0, g1, w0, w1):
        wid = lax.axis_index("s") * info.num_cores + lax.axis_index("c")
        base0 = wid * rpw
        pltpu.sync_copy(gidx_hbm.at[pl.ds(base0, rpw)], idx_all)
        bufs, gsem, wsem = (buf0, buf1), (g0, g1), (w0, w1)
        ghs = [None, None]
        whs = [None, None]
        for i in range(nch):
            p = i % 2
            if whs[p] is not None:
                whs[p].wait()
            ghs[p] = pltpu.async_copy(
                xkv_hbm.at[idx_all.at[pl.ds(i * _SC_CHUNK, _SC_CHUNK)]],
                bufs[p], gsem[p])
            q = 1 - p
            if ghs[q] is not None:
                ghs[q].wait()
                whs[q] = pltpu.async_copy(
                    bufs[q],
                    out_hbm.at[pl.ds(base0 + (i - 1) * _SC_CHUNK, _SC_CHUNK)],
                    wsem[q])
        pl2 = (nch - 1) % 2
        ghs[pl2].wait()
        pltpu.async_copy(
            bufs[pl2],
            out_hbm.at[pl.ds(base0 + (nch - 1) * _SC_CHUNK, _SC_CHUNK)],
            wsem[pl2]).wait()
        whs[1 - pl2].wait()

    return k(xkv, gidx)


_TNA = 256


def _attn_body(x_ref, xkv_ref, kvp_ref, wqT_ref, woT_ref, out_ref):
    xr = x_ref[:]
    q = jnp.dot(xr, wqT_ref[:], preferred_element_type=_F32, precision=_DEF)
    pk = kvp_ref[:]
    kp3 = lax.bitcast_convert_type(pk << 16, _F32)
    vp3 = lax.bitcast_convert_type(pk & jnp.int32(-65536), _F32)
    lane_h = lax.broadcasted_iota(jnp.int32, (_E, _H), 0) // _HD
    head_h = lax.broadcasted_iota(jnp.int32, (_E, _H), 1)
    hm = (lane_h == head_h).astype(_F32)
    lane_v = lax.broadcasted_iota(jnp.int32, (_H, _E), 1) // _HD
    head_v = lax.broadcasted_iota(jnp.int32, (_H, _E), 0)
    hmT = (lane_v == head_v).astype(_F32)

    s = kp3 * q[:, None, :]
    scores = jnp.dot(s.reshape(_TNA * _K, _E), hm,
                     preferred_element_type=_F32, precision=_DEF)
    scores = scores / np.sqrt(_HD).astype(np.float32)
    sc3 = scores.reshape(_TNA, _K, _H)
    mx = jnp.max(sc3, axis=1, keepdims=True)
    e = jnp.exp(sc3 - mx)
    a = e / jnp.sum(e, axis=1, keepdims=True)
    attn_exp = jnp.dot(a.reshape(_TNA * _K, _H), hmT,
                       preferred_element_type=_F32, precision=_DEF)
    o = jnp.sum(attn_exp.reshape(_TNA, _K, _E) * vp3, axis=1)
    o = o - lax.bitcast_convert_type(
        xkv_ref[:] & jnp.int32(-65536), _F32)
    out_ref[:] = xr + jnp.dot(o, woT_ref[:],
                              preferred_element_type=_F32, precision=_DEF)


def _attn_out(x2, xkv, kvp3, wqT, woutT):
    nrows = x2.shape[0]
    return pl.pallas_call(
        _attn_body,
        grid=(nrows // _TNA,),
        in_specs=[
            pl.BlockSpec((_TNA, _E), lambda i: (i, 0)),
            pl.BlockSpec((_TNA, _E), lambda i: (i, 0)),
            pl.BlockSpec((_TNA, _K, _E), lambda i: (i, 0, 0)),
            pl.BlockSpec((_E, _E), lambda i: (0, 0)),
            pl.BlockSpec((_E, _E), lambda i: (0, 0)),
        ],
        out_specs=pl.BlockSpec((_TNA, _E), lambda i: (i, 0)),
        out_shape=jax.ShapeDtypeStruct((nrows, _E), _F32),
    )(x2, xkv, kvp3, wqT, woutT)


def kernel(x, x_v, in_proj_weight, out_proj_weight):
    x2 = x.reshape(_B * _N, _E)
    xvp = jnp.pad(x_v, ((0, 0), (0, 0), (0, 128 - 3)))
    Wq, Wk, Wv = jnp.split(in_proj_weight, 3, axis=0)
    wqT = Wq.T
    wkvT = jnp.concatenate([Wk.T, Wv.T], axis=1)
    woutT = out_proj_weight.T

    gidxT = _knn_idx_global(xvp)
    gidx = jnp.transpose(gidxT, (0, 2, 1)).reshape(_ROWS)

    xkv = _proj_kv(x2, wkvT)
    kvp = _gather_rows_sc(xkv, gidx)
    out2 = _attn_out(x2, xkv, kvp.reshape(_B * _N, _K, _E), wqT, woutT)
    return out2.reshape(_B, _N, _E)


# speedup vs baseline: 1.2101x; 1.0646x over previous
import functools

import jax
import jax.numpy as jnp
import numpy as np
from jax import lax
from jax.experimental import pallas as pl
from jax.experimental.pallas import tpu as pltpu
from jax.experimental.pallas import tpu_sc as plsc

_B, _N, _E, _H, _K = 8, 2048, 256, 8, 8
_HD = _E // _H

_F32 = jnp.float32
_HIGH = lax.Precision.HIGHEST
_DEF = lax.Precision.DEFAULT

_TNB = 1024


def _knn_body(xvr_ref, xvc_ref, out_ref):
    b = pl.program_id(0)
    xr = xvr_ref[0]
    xc = xvc_ref[0]
    sqr = jnp.sum(xr * xr, axis=1, keepdims=True)
    ones8 = jnp.ones((8, 128), _F32)
    sqc = lax.dot_general(ones8, xc * xc, (((1,), (1,)), ((), ())),
                          preferred_element_type=_F32, precision=_HIGH)
    sqc_row = sqc[0:1, :]
    g = lax.dot_general(xr, xc, (((1,), (1,)), ((), ())),
                        preferred_element_type=_F32,
                        precision=lax.Precision.DEFAULT)
    d = (sqr + sqc_row) - 2.0 * g
    iota = lax.broadcasted_iota(jnp.int32, (_TNB, _N), 1)
    u = lax.bitcast_convert_type(jnp.maximum(d, 0.0), jnp.int32)
    key_i = (u & jnp.int32(~2047)) | iota
    key = lax.bitcast_convert_type(key_i + jnp.int32(0x00800000), _F32)
    for t in range(_K):
        m = jnp.min(key, axis=1, keepdims=True)
        mi = lax.bitcast_convert_type(m[:, 0], jnp.int32)
        out_ref[0, t, :] = (mi & 2047) + b * _N
        key = jnp.where(key == m, jnp.float32(jnp.inf), key)


def _knn_idx_global(xvp):
    return pl.pallas_call(
        _knn_body,
        grid=(_B, _N // _TNB),
        in_specs=[
            pl.BlockSpec((1, _TNB, 128), lambda b, i: (b, i, 0)),
            pl.BlockSpec((1, _N, 128), lambda b, i: (b, 0, 0)),
        ],
        out_specs=pl.BlockSpec((1, _K, _TNB), lambda b, i: (b, 0, i)),
        out_shape=jax.ShapeDtypeStruct((_B, _K, _N), jnp.int32),
    )(xvp, xvp)


_TNP = 512


def _proj_body(x_ref, w_ref, out_ref):
    pf = jnp.dot(x_ref[:], w_ref[:],
                 preferred_element_type=_F32, precision=_DEF)
    ki = lax.bitcast_convert_type(pf[:, :_E], jnp.int32)
    vi = lax.bitcast_convert_type(pf[:, _E:], jnp.int32)
    kr = ki + 0x7FFF + ((ki >> 16) & 1)
    vr = vi + 0x7FFF + ((vi >> 16) & 1)
    out_ref[:] = (lax.shift_right_logical(kr, 16) & 0xFFFF) | \
        (vr & jnp.int32(-65536))


def _proj_kv(x2, wkvT):
    return pl.pallas_call(
        _proj_body,
        grid=(_B * _N // _TNP,),
        in_specs=[
            pl.BlockSpec((_TNP, _E), lambda i: (i, 0)),
            pl.BlockSpec((_E, 2 * _E), lambda i: (0, 0)),
        ],
        out_specs=pl.BlockSpec((_TNP, _E), lambda i: (i, 0)),
        out_shape=jax.ShapeDtypeStruct((_B * _N, _E), jnp.int32),
    )(x2, wkvT)


_SC_CHUNK = 128
_ROWS = _B * _N * _K


def _gather_rows_sc(xkv, gidx):
    info = plsc.get_sparse_core_info()
    nw = info.num_cores * info.num_subcores
    rpw = _ROWS // nw
    nch = rpw // _SC_CHUNK
    mesh = plsc.VectorSubcoreMesh(core_axis_name="c", subcore_axis_name="s")

    @functools.partial(
        pl.kernel,
        mesh=mesh,
        out_type=jax.ShapeDtypeStruct((_ROWS, _E), jnp.int32),
        scratch_types=[
            pltpu.VMEM((rpw,), jnp.int32),
            pltpu.VMEM((_SC_CHUNK, _E), jnp.int32),
            pltpu.VMEM((_SC_CHUNK, _E), jnp.int32),
            pltpu.SemaphoreType.DMA,
            pltpu.SemaphoreType.DMA,
            pltpu.SemaphoreType.DMA,
            pltpu.SemaphoreType.DMA,
        ],
    )
    def k(xkv_hbm, gidx_hbm, out_hbm, idx_all, buf0, buf1,
          g0, g1, w0, w1):
        wid = lax.axis_index("s") * info.num_cores + lax.axis_index("c")
        base0 = wid * rpw
        pltpu.sync_copy(gidx_hbm.at[pl.ds(base0, rpw)], idx_all)
        bufs, gsem, wsem = (buf0, buf1), (g0, g1), (w0, w1)
        ghs = [None, None]
        whs = [None, None]
        for i in range(nch):
            p = i % 2
            if whs[p] is not None:
                whs[p].wait()
            ghs[p] = pltpu.async_copy(
                xkv_hbm.at[idx_all.at[pl.ds(i * _SC_CHUNK, _SC_CHUNK)]],
                bufs[p], gsem[p])
            q = 1 - p
            if ghs[q] is not None:
                ghs[q].wait()
                whs[q] = pltpu.async_copy(
                    bufs[q],
                    out_hbm.at[pl.ds(base0 + (i - 1) * _SC_CHUNK, _SC_CHUNK)],
                    wsem[q])
        pl2 = (nch - 1) % 2
        ghs[pl2].wait()
        pltpu.async_copy(
            bufs[pl2],
            out_hbm.at[pl.ds(base0 + (nch - 1) * _SC_CHUNK, _SC_CHUNK)],
            wsem[pl2]).wait()
        whs[1 - pl2].wait()

    return k(xkv, gidx)


_TNA = 256


def _attn_body(x_ref, xkv_ref, kvp_ref, wqT_ref, woT_ref, out_ref):
    xr = x_ref[:]
    q = jnp.dot(xr, wqT_ref[:], preferred_element_type=_F32, precision=_DEF)
    pk = kvp_ref[:]
    kp3 = lax.bitcast_convert_type(pk << 16, _F32)
    vp3 = lax.bitcast_convert_type(pk & jnp.int32(-65536), _F32)
    lane_h = lax.broadcasted_iota(jnp.int32, (_E, _H), 0) // _HD
    head_h = lax.broadcasted_iota(jnp.int32, (_E, _H), 1)
    hm = (lane_h == head_h).astype(_F32)
    lane_v = lax.broadcasted_iota(jnp.int32, (_H, _E), 1) // _HD
    head_v = lax.broadcasted_iota(jnp.int32, (_H, _E), 0)
    hmT = (lane_v == head_v).astype(_F32)

    s = kp3 * q[:, None, :]
    scores = jnp.dot(s.reshape(_TNA * _K, _E), hm,
                     preferred_element_type=_F32, precision=_DEF)
    scores = scores / np.sqrt(_HD).astype(np.float32)
    sc3 = scores.reshape(_TNA, _K, _H)
    mx = jnp.max(sc3, axis=1, keepdims=True)
    e = jnp.exp(sc3 - mx)
    a = e / jnp.sum(e, axis=1, keepdims=True)
    attn_exp = jnp.dot(a.reshape(_TNA * _K, _H), hmT,
                       preferred_element_type=_F32, precision=_DEF)
    o = jnp.sum(attn_exp.reshape(_TNA, _K, _E) * vp3, axis=1)
    o = o - lax.bitcast_convert_type(
        xkv_ref[:] & jnp.int32(-65536), _F32)
    out_ref[:] = xr + jnp.dot(o, woT_ref[:],
                              preferred_element_type=_F32, precision=_DEF)


def _attn_out(x2, xkv, kvp3, wqT, woutT):
    nrows = x2.shape[0]
    return pl.pallas_call(
        _attn_body,
        grid=(nrows // _TNA,),
        in_specs=[
            pl.BlockSpec((_TNA, _E), lambda i: (i, 0)),
            pl.BlockSpec((_TNA, _E), lambda i: (i, 0)),
            pl.BlockSpec((_TNA, _K, _E), lambda i: (i, 0, 0)),
            pl.BlockSpec((_E, _E), lambda i: (0, 0)),
            pl.BlockSpec((_E, _E), lambda i: (0, 0)),
        ],
        out_specs=pl.BlockSpec((_TNA, _E), lambda i: (i, 0)),
        out_shape=jax.ShapeDtypeStruct((nrows, _E), _F32),
    )(x2, xkv, kvp3, wqT, woutT)


def kernel(x, x_v, in_proj_weight, out_proj_weight):
    x2 = x.reshape(_B * _N, _E)
    xvp = jnp.pad(x_v, ((0, 0), (0, 0), (0, 128 - 3)))
    Wq, Wk, Wv = jnp.split(in_proj_weight, 3, axis=0)
    wqT = Wq.T
    wkvT = jnp.concatenate([Wk.T, Wv.T], axis=1)
    woutT = out_proj_weight.T

    gidxT = _knn_idx_global(xvp)
    gidx = jnp.transpose(gidxT, (0, 2, 1)).reshape(_ROWS)

    xkv = _proj_kv(x2, wkvT)
    kvp = _gather_rows_sc(xkv, gidx)
    out2 = _attn_out(x2, xkv, kvp.reshape(_B * _N, _K, _E), wqT, woutT)
    return out2.reshape(_B, _N, _E)


# speedup vs baseline: 1.2303x; 1.0167x over previous
import functools

import jax
import jax.numpy as jnp
import numpy as np
from jax import lax
from jax.experimental import pallas as pl
from jax.experimental.pallas import tpu as pltpu
from jax.experimental.pallas import tpu_sc as plsc

_B, _N, _E, _H, _K = 8, 2048, 256, 8, 8
_HD = _E // _H

_F32 = jnp.float32
_HIGH = lax.Precision.HIGHEST
_DEF = lax.Precision.DEFAULT

_TNB = 1024


def _knn_body(xvr_ref, xvc_ref, out_ref):
    b = pl.program_id(0)
    xr = xvr_ref[0]
    xc = xvc_ref[0]
    sqr = jnp.sum(xr * xr, axis=1, keepdims=True)
    ones8 = jnp.ones((8, 128), _F32)
    sqc = lax.dot_general(ones8, xc * xc, (((1,), (1,)), ((), ())),
                          preferred_element_type=_F32, precision=_HIGH)
    sqc_row = sqc[0:1, :]
    g = lax.dot_general(xr, xc, (((1,), (1,)), ((), ())),
                        preferred_element_type=_F32,
                        precision=lax.Precision.DEFAULT)
    d = (sqr + sqc_row) - 2.0 * g
    iota = lax.broadcasted_iota(jnp.int32, (_TNB, _N), 1)
    u = lax.bitcast_convert_type(jnp.maximum(d, 0.0), jnp.int32)
    key_i = (u & jnp.int32(~2047)) | iota
    key = lax.bitcast_convert_type(key_i + jnp.int32(0x00800000), _F32)
    for t in range(_K):
        m = jnp.min(key, axis=1, keepdims=True)
        mi = lax.bitcast_convert_type(m[:, 0], jnp.int32)
        out_ref[0, t, :] = (mi & 2047) + b * _N
        key = jnp.where(key == m, jnp.float32(jnp.inf), key)


def _knn_idx_global(xvp):
    return pl.pallas_call(
        _knn_body,
        grid=(_B, _N // _TNB),
        in_specs=[
            pl.BlockSpec((1, _TNB, 128), lambda b, i: (b, i, 0)),
            pl.BlockSpec((1, _N, 128), lambda b, i: (b, 0, 0)),
        ],
        out_specs=pl.BlockSpec((1, _K, _TNB), lambda b, i: (b, 0, i)),
        out_shape=jax.ShapeDtypeStruct((_B, _K, _N), jnp.int32),
    )(xvp, xvp)


_TNP = 512


def _proj_body(x_ref, w_ref, out_ref):
    pf = jnp.dot(x_ref[:], w_ref[:],
                 preferred_element_type=_F32, precision=_DEF)
    ki = lax.bitcast_convert_type(pf[:, :_E], jnp.int32)
    vi = lax.bitcast_convert_type(pf[:, _E:], jnp.int32)
    kr = ki + 0x7FFF + ((ki >> 16) & 1)
    vr = vi + 0x7FFF + ((vi >> 16) & 1)
    out_ref[:] = (lax.shift_right_logical(kr, 16) & 0xFFFF) | \
        (vr & jnp.int32(-65536))


def _proj_kv(x2, wkvT):
    return pl.pallas_call(
        _proj_body,
        grid=(_B * _N // _TNP,),
        in_specs=[
            pl.BlockSpec((_TNP, _E), lambda i: (i, 0)),
            pl.BlockSpec((_E, 2 * _E), lambda i: (0, 0)),
        ],
        out_specs=pl.BlockSpec((_TNP, _E), lambda i: (i, 0)),
        out_shape=jax.ShapeDtypeStruct((_B * _N, _E), jnp.int32),
    )(x2, wkvT)


_SC_CHUNK = 128
_ROWS = _B * _N * _K


def _gather_rows_sc(xkv, gidx):
    info = plsc.get_sparse_core_info()
    nw = info.num_cores * info.num_subcores
    rpw = _ROWS // nw
    nch = rpw // _SC_CHUNK
    mesh = plsc.VectorSubcoreMesh(core_axis_name="c", subcore_axis_name="s")

    @functools.partial(
        pl.kernel,
        mesh=mesh,
        out_type=jax.ShapeDtypeStruct((_ROWS, _E), jnp.int32),
        scratch_types=[
            pltpu.VMEM((rpw,), jnp.int32),
            pltpu.VMEM((_SC_CHUNK, _E), jnp.int32),
            pltpu.VMEM((_SC_CHUNK, _E), jnp.int32),
            pltpu.SemaphoreType.DMA,
            pltpu.SemaphoreType.DMA,
            pltpu.SemaphoreType.DMA,
            pltpu.SemaphoreType.DMA,
        ],
    )
    def k(xkv_hbm, gidx_hbm, out_hbm, idx_all, buf0, buf1,
          g0, g1, w0, w1):
        wid = lax.axis_index("s") * info.num_cores + lax.axis_index("c")
        base0 = wid * rpw
        pltpu.sync_copy(gidx_hbm.at[pl.ds(base0, rpw)], idx_all)
        bufs, gsem, wsem = (buf0, buf1), (g0, g1), (w0, w1)
        ghs = [None, None]
        whs = [None, None]
        for i in range(nch):
            p = i % 2
            if whs[p] is not None:
                whs[p].wait()
            ghs[p] = pltpu.async_copy(
                xkv_hbm.at[idx_all.at[pl.ds(i * _SC_CHUNK, _SC_CHUNK)]],
                bufs[p], gsem[p])
            q = 1 - p
            if ghs[q] is not None:
                ghs[q].wait()
                whs[q] = pltpu.async_copy(
                    bufs[q],
                    out_hbm.at[pl.ds(base0 + (i - 1) * _SC_CHUNK, _SC_CHUNK)],
                    wsem[q])
        pl2 = (nch - 1) % 2
        ghs[pl2].wait()
        pltpu.async_copy(
            bufs[pl2],
            out_hbm.at[pl.ds(base0 + (nch - 1) * _SC_CHUNK, _SC_CHUNK)],
            wsem[pl2]).wait()
        whs[1 - pl2].wait()

    return k(xkv, gidx)


_TNA = 512


def _attn_body(x_ref, xkv_ref, kvp_ref, wqT_ref, woT_ref, out_ref):
    xr = x_ref[:]
    q = jnp.dot(xr, wqT_ref[:], preferred_element_type=_F32, precision=_DEF)
    pk = kvp_ref[:]
    kp3 = lax.bitcast_convert_type(pk << 16, _F32)
    vp3 = lax.bitcast_convert_type(pk & jnp.int32(-65536), _F32)
    lane_h = lax.broadcasted_iota(jnp.int32, (_E, _H), 0) // _HD
    head_h = lax.broadcasted_iota(jnp.int32, (_E, _H), 1)
    hm = (lane_h == head_h).astype(_F32)
    lane_v = lax.broadcasted_iota(jnp.int32, (_H, _E), 1) // _HD
    head_v = lax.broadcasted_iota(jnp.int32, (_H, _E), 0)
    hmT = (lane_v == head_v).astype(_F32)

    s = kp3 * q[:, None, :]
    scores = jnp.dot(s.reshape(_TNA * _K, _E), hm,
                     preferred_element_type=_F32, precision=_DEF)
    scores = scores / np.sqrt(_HD).astype(np.float32)
    sc3 = scores.reshape(_TNA, _K, _H)
    mx = jnp.max(sc3, axis=1, keepdims=True)
    e = jnp.exp(sc3 - mx)
    a = e / jnp.sum(e, axis=1, keepdims=True)
    attn_exp = jnp.dot(a.reshape(_TNA * _K, _H), hmT,
                       preferred_element_type=_F32, precision=_DEF)
    o = jnp.sum(attn_exp.reshape(_TNA, _K, _E) * vp3, axis=1)
    o = o - lax.bitcast_convert_type(
        xkv_ref[:] & jnp.int32(-65536), _F32)
    out_ref[:] = xr + jnp.dot(o, woT_ref[:],
                              preferred_element_type=_F32, precision=_DEF)


def _attn_out(x2, xkv, kvp3, wqT, woutT):
    nrows = x2.shape[0]
    return pl.pallas_call(
        _attn_body,
        grid=(nrows // _TNA,),
        in_specs=[
            pl.BlockSpec((_TNA, _E), lambda i: (i, 0)),
            pl.BlockSpec((_TNA, _E), lambda i: (i, 0)),
            pl.BlockSpec((_TNA, _K, _E), lambda i: (i, 0, 0)),
            pl.BlockSpec((_E, _E), lambda i: (0, 0)),
            pl.BlockSpec((_E, _E), lambda i: (0, 0)),
        ],
        out_specs=pl.BlockSpec((_TNA, _E), lambda i: (i, 0)),
        out_shape=jax.ShapeDtypeStruct((nrows, _E), _F32),
    )(x2, xkv, kvp3, wqT, woutT)


def kernel(x, x_v, in_proj_weight, out_proj_weight):
    x2 = x.reshape(_B * _N, _E)
    xvp = jnp.pad(x_v, ((0, 0), (0, 0), (0, 128 - 3)))
    Wq, Wk, Wv = jnp.split(in_proj_weight, 3, axis=0)
    wqT = Wq.T
    wkvT = jnp.concatenate([Wk.T, Wv.T], axis=1)
    woutT = out_proj_weight.T

    gidxT = _knn_idx_global(xvp)
    gidx = jnp.transpose(gidxT, (0, 2, 1)).reshape(_ROWS)

    xkv = _proj_kv(x2, wkvT)
    kvp = _gather_rows_sc(xkv, gidx)
    out2 = _attn_out(x2, xkv, kvp.reshape(_B * _N, _K, _E), wqT, woutT)
    return out2.reshape(_B, _N, _E)
